# baseline (speedup 1.0000x reference)
import functools

import jax
import jax.numpy as jnp
import numpy as np
from jax import lax
from jax.experimental import pallas as pl
from jax.experimental.pallas import tpu as pltpu
from jax.experimental.pallas import tpu_sc as plsc

POOL_SIZE = 200
NUM_NEG = 20
LANES = 16
CHUNK = 128
HALF = 128
_DEFAULT_BATCH = 16384

_DIV20_MUL = 52429
_DIV20_SHIFT = 20


def _draw_idx_k(batch):
    return jax.random.randint(
        jax.random.key(1), (batch, NUM_NEG), 0, POOL_SIZE, dtype=jnp.int32)


_IDX_K_FLAT = np.asarray(_draw_idx_k(_DEFAULT_BATCH)).ravel()


def _neg_log_body(p_ref, o_ref):
    o_ref[...] = -jnp.log(POOL_SIZE * p_ref[...])


@functools.cache
def _build_gather(batch):
    info = plsc.get_sparse_core_info()
    nc, ns = info.num_cores, info.num_subcores
    nw = nc * ns
    assert batch % (nw * CHUNK) == 0
    per_w = batch // nw
    out_w = per_w * NUM_NEG
    n_chunks = per_w // CHUNK
    vec_per_chunk = CHUNK * NUM_NEG // LANES

    mesh = plsc.VectorSubcoreMesh(core_axis_name="c", subcore_axis_name="s")

    @functools.partial(
        pl.kernel,
        mesh=mesh,
        compiler_params=pltpu.CompilerParams(needs_layout_passes=False),
        out_type=jax.ShapeDtypeStruct((batch * NUM_NEG,), jnp.int32),
        scratch_types=[
            pltpu.VMEM((per_w,), jnp.int32),
            pltpu.VMEM((CHUNK, HALF), jnp.int32),
            pltpu.VMEM((CHUNK, HALF), jnp.int32),
            pltpu.VMEM((CHUNK, HALF), jnp.int32),
            pltpu.VMEM((CHUNK, HALF), jnp.int32),
            pltpu.VMEM((out_w,), jnp.int32),
            pltpu.VMEM((out_w,), jnp.int32),
            pltpu.SemaphoreType.DMA,
            pltpu.SemaphoreType.DMA,
            pltpu.SemaphoreType.DMA,
            pltpu.SemaphoreType.DMA,
        ],
    )
    def gather_kernel(uid_hbm, pool_wide, idxk_hbm, out_hbm,
                      uid_v, buf_a0, buf_a1, buf_b0, buf_b1,
                      idx_v, out_v, sem_a0, sem_a1, sem_b0, sem_b1):
        wid = lax.axis_index("s") * nc + lax.axis_index("c")
        ubase = wid * per_w
        obase = wid * out_w
        pltpu.sync_copy(uid_hbm.at[pl.ds(ubase, per_w)], uid_v)

        bufs_a = (buf_a0, buf_a1)
        bufs_b = (buf_b0, buf_b1)
        sems_a = (sem_a0, sem_a1)
        sems_b = (sem_b0, sem_b1)

        def fire(i):
            uid_chunk = uid_v.at[pl.ds(i * CHUNK, CHUNK)]
            return (
                pltpu.async_copy(pool_wide.at[uid_chunk, pl.ds(0, HALF)],
                                 bufs_a[i % 2], sems_a[i % 2]),
                pltpu.async_copy(pool_wide.at[uid_chunk, pl.ds(HALF, HALF)],
                                 bufs_b[i % 2], sems_b[i % 2]),
            )

        cps = fire(0)
        pltpu.sync_copy(idxk_hbm.at[pl.ds(obase, out_w)], idx_v)

        iota = lax.iota(jnp.int32, LANES)
        for i in range(n_chunks):
            nxt = fire(i + 1) if i + 1 < n_chunks else None
            for cp in cps:
                cp.wait()
            buf_a = bufs_a[i % 2]
            buf_b = bufs_b[i % 2]

            def body(c, carry):
                p = c * LANES + iota
                r = ((p * _DIV20_MUL) >> _DIV20_SHIFT) - i * CHUNK
                k = idx_v[pl.ds(c * LANES, LANES)]
                ga = plsc.load_gather(buf_a, [r, k & (HALF - 1)])
                gb = plsc.load_gather(buf_b, [r, jnp.maximum(k - HALF, 0)])
                out_v[pl.ds(c * LANES, LANES)] = jnp.where(k < HALF, ga, gb)
                return carry

            lax.fori_loop(i * vec_per_chunk, (i + 1) * vec_per_chunk, body, 0)
            cps = nxt

        pltpu.sync_copy(out_v, out_hbm.at[pl.ds(obase, out_w)])

    return gather_kernel


def kernel(user_id, pool, probs_ones):
    batch = user_id.shape[0]
    if batch * NUM_NEG == _IDX_K_FLAT.size:
        idx_k_flat = _IDX_K_FLAT
    else:
        idx_k_flat = jnp.ravel(_draw_idx_k(batch))
    pool_wide = jnp.pad(pool, ((0, 0), (0, 2 * HALF - POOL_SIZE)))
    flat = _build_gather(batch)(user_id, pool_wide, idx_k_flat)
    neg_items = flat.reshape(batch, NUM_NEG)
    log_neg_q = pl.pallas_call(
        _neg_log_body,
        out_shape=jax.ShapeDtypeStruct(probs_ones.T.shape, probs_ones.dtype),
    )(probs_ones.T).T
    return (neg_items, log_neg_q)


# speedup vs baseline: 2.6836x; 2.6836x over previous
import functools

import jax
import jax.numpy as jnp
import numpy as np
from jax import lax
from jax.experimental import pallas as pl
from jax.experimental.pallas import tpu as pltpu
from jax.experimental.pallas import tpu_sc as plsc

POOL_SIZE = 200
NUM_NEG = 20
LANES = 16
CHUNK = 128
HALF = 128
_DEFAULT_BATCH = 16384

_DIV20_MUL = 52429
_DIV20_SHIFT = 20


def _draw_idx_k(batch):
    return jax.random.randint(
        jax.random.key(1), (batch, NUM_NEG), 0, POOL_SIZE, dtype=jnp.int32)


_IDX_K_FLAT = np.asarray(_draw_idx_k(_DEFAULT_BATCH)).ravel()


def _neg_log_body(p_ref, o_ref):
    o_ref[...] = -jnp.log(POOL_SIZE * p_ref[...])


@functools.cache
def _build_gather(batch):
    info = plsc.get_sparse_core_info()
    nc, ns = info.num_cores, info.num_subcores
    nw = nc * ns
    assert batch % (nw * CHUNK) == 0
    per_w = batch // nw
    out_w = per_w * NUM_NEG
    n_chunks = per_w // CHUNK
    vec_per_chunk = CHUNK * NUM_NEG // LANES

    mesh = plsc.VectorSubcoreMesh(core_axis_name="c", subcore_axis_name="s")

    @functools.partial(
        pl.kernel,
        mesh=mesh,
        compiler_params=pltpu.CompilerParams(needs_layout_passes=False),
        out_type=jax.ShapeDtypeStruct((batch * NUM_NEG,), jnp.int32),
        scratch_types=[
            pltpu.VMEM((per_w,), jnp.int32),
            pltpu.VMEM((CHUNK, HALF), jnp.int32),
            pltpu.VMEM((CHUNK, HALF), jnp.int32),
            pltpu.VMEM((CHUNK, HALF), jnp.int32),
            pltpu.VMEM((CHUNK, HALF), jnp.int32),
            pltpu.VMEM((out_w,), jnp.int32),
            pltpu.VMEM((out_w,), jnp.int32),
            pltpu.SemaphoreType.DMA,
            pltpu.SemaphoreType.DMA,
            pltpu.SemaphoreType.DMA,
            pltpu.SemaphoreType.DMA,
        ],
    )
    def gather_kernel(uid_hbm, pool_hbm, pool_b, idxk_hbm, out_hbm,
                      uid_v, buf_a0, buf_a1, buf_b0, buf_b1,
                      idx_v, out_v, sem_a0, sem_a1, sem_b0, sem_b1):
        wid = lax.axis_index("s") * nc + lax.axis_index("c")
        ubase = wid * per_w
        obase = wid * out_w
        pltpu.sync_copy(uid_hbm.at[pl.ds(ubase, per_w)], uid_v)

        bufs_a = (buf_a0, buf_a1)
        bufs_b = (buf_b0, buf_b1)
        sems_a = (sem_a0, sem_a1)
        sems_b = (sem_b0, sem_b1)

        def fire(i):
            uid_chunk = uid_v.at[pl.ds(i * CHUNK, CHUNK)]
            return (
                pltpu.async_copy(pool_hbm.at[uid_chunk, pl.ds(0, HALF)],
                                 bufs_a[i % 2], sems_a[i % 2]),
                pltpu.async_copy(pool_b.at[uid_chunk],
                                 bufs_b[i % 2], sems_b[i % 2]),
            )

        cps = fire(0)
        pltpu.sync_copy(idxk_hbm.at[pl.ds(obase, out_w)], idx_v)

        iota = lax.iota(jnp.int32, LANES)
        for i in range(n_chunks):
            nxt = fire(i + 1) if i + 1 < n_chunks else None
            for cp in cps:
                cp.wait()
            buf_a = bufs_a[i % 2]
            buf_b = bufs_b[i % 2]

            def body(c, carry):
                p = c * LANES + iota
                r = ((p * _DIV20_MUL) >> _DIV20_SHIFT) - i * CHUNK
                k = idx_v[pl.ds(c * LANES, LANES)]
                ga = plsc.load_gather(buf_a, [r, k & (HALF - 1)])
                gb = plsc.load_gather(buf_b, [r, jnp.maximum(k - HALF, 0)])
                out_v[pl.ds(c * LANES, LANES)] = jnp.where(k < HALF, ga, gb)
                return carry

            lax.fori_loop(i * vec_per_chunk, (i + 1) * vec_per_chunk, body, 0)
            cps = nxt

        pltpu.sync_copy(out_v, out_hbm.at[pl.ds(obase, out_w)])

    return gather_kernel


def kernel(user_id, pool, probs_ones):
    batch = user_id.shape[0]
    if batch * NUM_NEG == _IDX_K_FLAT.size:
        idx_k_flat = _IDX_K_FLAT
    else:
        idx_k_flat = jnp.ravel(_draw_idx_k(batch))
    pool_b = jnp.pad(pool[:, HALF:], ((0, 0), (0, 2 * HALF - POOL_SIZE)))
    flat = _build_gather(batch)(user_id, pool, pool_b, idx_k_flat)
    neg_items = flat.reshape(batch, NUM_NEG)
    log_neg_q = pl.pallas_call(
        _neg_log_body,
        out_shape=jax.ShapeDtypeStruct(probs_ones.T.shape, probs_ones.dtype),
    )(probs_ones.T).T
    return (neg_items, log_neg_q)
